# baseline (speedup 1.0000x reference)
import functools

import jax
import jax.numpy as jnp
from jax import lax
from jax.experimental import pallas as pl
from jax.experimental.pallas import tpu as pltpu
from jax.experimental.pallas import tpu_sc as plsc

PH = 16
DH = 1024
NROWS = PH * PH
D = 2 * DH
NC = 2
NS = 16
RPW = NROWS // (NC * NS)

_mesh = plsc.VectorSubcoreMesh(core_axis_name="c", subcore_axis_name="s")


@functools.partial(
    pl.kernel,
    mesh=_mesh,
    out_type=jax.ShapeDtypeStruct((NROWS, D), jnp.float32),
    scratch_types=[
        pltpu.VMEM((1, DH), jnp.float32),
        pltpu.VMEM((RPW, DH), jnp.float32),
        pltpu.SemaphoreType.DMA,
    ],
)
def _spatial_pos_enc(row_hbm, col_hbm, out_hbm, r_v, c_v, sem):
    wid = lax.axis_index("s") * NC + lax.axis_index("c")
    i = wid // 2
    j0 = (wid % 2) * RPW
    base = wid * RPW

    in_r = pltpu.async_copy(row_hbm.at[pl.ds(i, 1)], r_v, sem)
    in_c = pltpu.async_copy(col_hbm.at[pl.ds(j0, RPW)], c_v, sem)
    in_r.wait()
    in_c.wait()

    outs = [pltpu.async_copy(c_v, out_hbm.at[pl.ds(base, RPW), pl.ds(DH, DH)], sem)]
    for t in range(RPW):
        outs.append(
            pltpu.async_copy(r_v, out_hbm.at[pl.ds(base + t, 1), pl.ds(0, DH)], sem)
        )
    for h in outs:
        h.wait()


def kernel(row_embed, col_embed):
    return _spatial_pos_enc(row_embed, col_embed)
